# Initial kernel scaffold; baseline (speedup 1.0000x reference)
import jax
import jax.numpy as jnp
from jax.experimental import pallas as pl


def kernel(x, edge_index, W_feat, b_feat, W_pool, b_pool):
    raise NotImplementedError("write your pallas kernel here")


# speedup vs baseline: 1.7973x; 1.7973x over previous
import jax
import jax.numpy as jnp
from jax.experimental import pallas as pl

N = 10000
E = 320000
D = 128
ASSIGN_DIM = 1250
BATCH = 50
NPG = N // BATCH
CPG = ASSIGN_DIM // BATCH


def _fused_graphsage_kernel(x_ref, c_ref, wf_ref, bf_ref, wp_ref, bp_ref,
                            feat_ref, s_ref):
    g = pl.program_id(0)
    x = x_ref[...]
    c = c_ref[...]
    wf = wf_ref[...]
    wp = wp_ref[...]

    fp = (jnp.dot(x, wf[:D], preferred_element_type=jnp.float32)
          + jnp.dot(c, wf[D:], preferred_element_type=jnp.float32)
          + bf_ref[...])
    fn = jnp.sqrt(jnp.sum(fp * fp, axis=1, keepdims=True))
    feat_ref[...] = jnp.maximum(fp / jnp.maximum(fn, 1e-12), 0.0)

    ap = (jnp.dot(x, wp[:D], preferred_element_type=jnp.float32)
          + jnp.dot(c, wp[D:], preferred_element_type=jnp.float32)
          + bp_ref[...])
    an = jnp.sqrt(jnp.sum(ap * ap, axis=1, keepdims=True))
    a = jnp.maximum(ap / jnp.maximum(an, 1e-12), 0.0)

    cols = jax.lax.broadcasted_iota(jnp.int32, (ASSIGN_DIM, CPG), 0)
    ks = jax.lax.broadcasted_iota(jnp.int32, (ASSIGN_DIM, CPG), 1)
    sel = (cols == g * CPG + ks).astype(jnp.float32)
    ab = jnp.dot(a, sel, preferred_element_type=jnp.float32)

    m = jnp.max(ab, axis=1, keepdims=True)
    eb = jnp.exp(ab - m)
    sb = jnp.sum(eb, axis=1, keepdims=True)
    z = sb + (ASSIGN_DIM - CPG) * jnp.exp(-m)
    s_ref[...] = eb / (sb + 1e-13 * z)


def _pool_kernel(s_ref, feat_ref, as_ref, h_ref, adj_ref):
    s = s_ref[...]
    st_feat = jax.lax.dot_general(
        s, feat_ref[...], (((0,), (0,)), ((), ())),
        preferred_element_type=jnp.float32)
    h_ref[...] = st_feat[None]
    adj_ref[...] = jax.lax.dot_general(
        s, as_ref[...], (((0,), (0,)), ((), ())),
        preferred_element_type=jnp.float32)[None]


def kernel(x, edge_index, W_feat, b_feat, W_pool, b_pool):
    src = edge_index[0]
    dst = edge_index[1]

    msg = jax.ops.segment_sum(x[src], dst, num_segments=N)
    deg = jax.ops.segment_sum(jnp.ones((E,), x.dtype), dst, num_segments=N)
    c = msg / jnp.maximum(deg, 1.0)[:, None]

    feat, s_c = pl.pallas_call(
        _fused_graphsage_kernel,
        grid=(BATCH,),
        in_specs=[
            pl.BlockSpec((NPG, D), lambda g: (g, 0)),
            pl.BlockSpec((NPG, D), lambda g: (g, 0)),
            pl.BlockSpec((2 * D, D), lambda g: (0, 0)),
            pl.BlockSpec((1, D), lambda g: (0, 0)),
            pl.BlockSpec((2 * D, ASSIGN_DIM), lambda g: (0, 0)),
            pl.BlockSpec((1, ASSIGN_DIM), lambda g: (0, 0)),
        ],
        out_specs=[
            pl.BlockSpec((NPG, D), lambda g: (g, 0)),
            pl.BlockSpec((NPG, CPG), lambda g: (g, 0)),
        ],
        out_shape=[
            jax.ShapeDtypeStruct((N, D), jnp.float32),
            jax.ShapeDtypeStruct((N, CPG), jnp.float32),
        ],
    )(x, c, W_feat, b_feat.reshape(1, D), W_pool, b_pool.reshape(1, ASSIGN_DIM))

    seg = dst * BATCH + src // NPG
    adj_s = jax.ops.segment_sum(s_c[src], seg, num_segments=N * BATCH)
    adj_s = adj_s.reshape(N, ASSIGN_DIM)

    h_new, adj_new = pl.pallas_call(
        _pool_kernel,
        grid=(BATCH,),
        in_specs=[
            pl.BlockSpec((NPG, CPG), lambda g: (g, 0)),
            pl.BlockSpec((NPG, D), lambda g: (g, 0)),
            pl.BlockSpec((NPG, ASSIGN_DIM), lambda g: (g, 0)),
        ],
        out_specs=[
            pl.BlockSpec((1, CPG, D), lambda g: (g, 0, 0)),
            pl.BlockSpec((1, CPG, ASSIGN_DIM), lambda g: (g, 0, 0)),
        ],
        out_shape=[
            jax.ShapeDtypeStruct((BATCH, CPG, D), jnp.float32),
            jax.ShapeDtypeStruct((BATCH, CPG, ASSIGN_DIM), jnp.float32),
        ],
    )(s_c, feat, adj_s)

    return (adj_new.reshape(ASSIGN_DIM, ASSIGN_DIM),
            h_new.reshape(ASSIGN_DIM, D))
